# baseline (speedup 1.0000x reference)
import functools

import jax
import jax.numpy as jnp
import numpy as np
from jax import lax
from jax.experimental import pallas as pl
from jax.experimental.pallas import tpu as pltpu
from jax.experimental.pallas import tpu_sc as plsc

_VOCAB = 100000
_EMBED = 128
_DMODEL = 768
_MAXLEN = 2048
_EPS = 1e-5
_BLK = 2048


def _sinusoidal_pe_np(max_len, d):
    pos = np.arange(max_len, dtype=np.float32)[:, None]
    div = np.exp(np.arange(0, d, 2, dtype=np.float32) * (-np.log(10000.0) / d))
    pe = np.zeros((max_len, d), dtype=np.float32)
    pe[:, 0::2] = np.sin(pos * div)
    pe[:, 1::2] = np.cos(pos * div)
    return pe


def _sc_gather(token_table, sequence, n_rows, s_len):
    info = plsc.get_sparse_core_info()
    nc, ns = info.num_cores, info.num_subcores
    nw = nc * ns
    rows_per_w = n_rows // nw
    sub = rows_per_w // 128

    mesh = plsc.VectorSubcoreMesh(core_axis_name="c", subcore_axis_name="s")

    @functools.partial(
        pl.kernel,
        mesh=mesh,
        out_type=jax.ShapeDtypeStruct((n_rows, _EMBED), jnp.float32),
        scratch_types=[
            pltpu.VMEM((sub, 128), jnp.int32),
            pltpu.VMEM((rows_per_w, _EMBED), jnp.float32),
            pltpu.SemaphoreType.DMA,
        ],
    )
    def gather_kernel(table_hbm, seq_hbm, out_hbm, idx_v, rows_v, sem):
        wid = lax.axis_index("s") * nc + lax.axis_index("c")
        flat = wid * rows_per_w
        batch = flat // s_len
        col = flat % s_len
        for j in range(sub):
            pltpu.sync_copy(seq_hbm.at[batch, pl.ds(col + j * 128, 128)],
                            idx_v.at[j])
        copies = [
            pltpu.async_copy(table_hbm.at[idx_v.at[j]],
                             rows_v.at[pl.ds(j * 128, 128)], sem)
            for j in range(sub)
        ]
        for c in copies:
            c.wait()
        pltpu.sync_copy(rows_v, out_hbm.at[pl.ds(wid * rows_per_w, rows_per_w)])

    return gather_kernel(token_table, sequence)


def _tc_body(g_ref, pe_ref, seg_ref, segtab_ref, w_ref, bgb_ref, out_ref):
    j = pl.program_id(0)
    seg_row = seg_ref[pl.ds(j, 1), :]
    rid = lax.broadcasted_iota(jnp.int32, (8, 1), 0)
    onehot_t = jnp.where(seg_row == rid, 1.0, 0.0)
    seg_emb = lax.dot_general(
        onehot_t, segtab_ref[...],
        dimension_numbers=(((0,), (0,)), ((), ())),
        precision=lax.Precision.HIGHEST,
        preferred_element_type=jnp.float32)
    x = g_ref[...] + pe_ref[...] + seg_emb
    y = jnp.dot(x, w_ref[...], preferred_element_type=jnp.float32)
    y = y + bgb_ref[0, :][None, :]
    mu = jnp.mean(y, axis=-1, keepdims=True)
    d = y - mu
    var = jnp.mean(d * d, axis=-1, keepdims=True)
    yn = d * lax.rsqrt(var + _EPS)
    out_ref[...] = yn * bgb_ref[1, :][None, :] + bgb_ref[2, :][None, :]


def _tc_fused(g, pe, seg, segtab, W, bgb, bsz, n_rows, s_len):
    return pl.pallas_call(
        _tc_body,
        grid=(n_rows // _BLK,),
        in_specs=[
            pl.BlockSpec((_BLK, _EMBED), lambda j: (j, 0)),
            pl.BlockSpec((s_len, _EMBED), lambda j: (0, 0)),
            pl.BlockSpec((bsz, s_len), lambda j: (0, 0)),
            pl.BlockSpec((8, _EMBED), lambda j: (0, 0)),
            pl.BlockSpec((_EMBED, _DMODEL), lambda j: (0, 0)),
            pl.BlockSpec((3, _DMODEL), lambda j: (0, 0)),
        ],
        out_specs=pl.BlockSpec((_BLK, _DMODEL), lambda j: (j, 0)),
        out_shape=jax.ShapeDtypeStruct((n_rows, _DMODEL), jnp.float32),
    )(g, pe, seg, segtab, W, bgb)


def kernel(sequence, sequence_segment, token_table, seg_table, W, b, gamma, beta):
    bsz, s_len = sequence.shape
    n_rows = bsz * s_len

    g = _sc_gather(token_table, sequence.astype(jnp.int32), n_rows, s_len)

    pe = jnp.asarray(_sinusoidal_pe_np(_MAXLEN, _EMBED)[:s_len])
    bgb = jnp.stack([b, gamma, beta])

    segtab_pad = jnp.zeros((8, _EMBED), jnp.float32).at[:3].set(seg_table)
    out = _tc_fused(g, pe, sequence_segment.astype(jnp.int32), segtab_pad, W,
                    bgb, bsz, n_rows, s_len)
    return jnp.reshape(out, (bsz, s_len, _DMODEL))


# speedup vs baseline: 1.1765x; 1.1765x over previous
import functools

import jax
import jax.numpy as jnp
import numpy as np
from jax import lax
from jax.experimental import pallas as pl
from jax.experimental.pallas import tpu as pltpu
from jax.experimental.pallas import tpu_sc as plsc

_VOCAB = 100000
_EMBED = 128
_DMODEL = 768
_MAXLEN = 2048
_EPS = 1e-5
_BLK = 2048


def _sinusoidal_pe_np(max_len, d):
    pos = np.arange(max_len, dtype=np.float32)[:, None]
    div = np.exp(np.arange(0, d, 2, dtype=np.float32) * (-np.log(10000.0) / d))
    pe = np.zeros((max_len, d), dtype=np.float32)
    pe[:, 0::2] = np.sin(pos * div)
    pe[:, 1::2] = np.cos(pos * div)
    return pe


def _sc_gather(token_table, sequence, n_rows, s_len):
    info = plsc.get_sparse_core_info()
    nc, ns = info.num_cores, info.num_subcores
    nw = nc * ns
    rows_per_w = n_rows // nw
    sub = rows_per_w // 128

    mesh = plsc.VectorSubcoreMesh(core_axis_name="c", subcore_axis_name="s")

    @functools.partial(
        pl.kernel,
        mesh=mesh,
        out_type=jax.ShapeDtypeStruct((n_rows, _EMBED), jnp.float32),
        scratch_types=[
            pltpu.VMEM((sub, 128), jnp.int32),
            pltpu.VMEM((rows_per_w, _EMBED), jnp.float32),
            pltpu.SemaphoreType.DMA,
        ],
    )
    def gather_kernel(table_hbm, seq_hbm, out_hbm, idx_v, rows_v, sem):
        wid = lax.axis_index("s") * nc + lax.axis_index("c")
        flat = wid * rows_per_w
        batch = flat // s_len
        col = flat % s_len
        for j in range(sub):
            pltpu.sync_copy(seq_hbm.at[batch, pl.ds(col + j * 128, 128)],
                            idx_v.at[j])
        copies = [
            pltpu.async_copy(table_hbm.at[idx_v.at[j]],
                             rows_v.at[pl.ds(j * 128, 128)], sem)
            for j in range(sub)
        ]
        for c in copies:
            c.wait()
        pltpu.sync_copy(rows_v, out_hbm.at[pl.ds(wid * rows_per_w, rows_per_w)])

    return gather_kernel(token_table, sequence)


def _tc_body(g_ref, pe_ref, seg_ref, segtab_ref, w_ref, bgb_ref, out_ref):
    x = g_ref[...] + pe_ref[...]
    seg = seg_ref[...].astype(jnp.int32)
    for r in range(3):
        mask = jnp.where(seg == r, 1.0, 0.0)
        x = x + mask * segtab_ref[r, :][None, :]
    y = jnp.dot(x, w_ref[...], preferred_element_type=jnp.float32)
    y = y + bgb_ref[0, :][None, :]
    mu = jnp.mean(y, axis=-1, keepdims=True)
    d = y - mu
    var = jnp.mean(d * d, axis=-1, keepdims=True)
    yn = d * lax.rsqrt(var + _EPS)
    out_ref[...] = yn * bgb_ref[1, :][None, :] + bgb_ref[2, :][None, :]


def _tc_fused(g, pe, seg, segtab, W, bgb, bsz, n_rows, s_len):
    return pl.pallas_call(
        _tc_body,
        grid=(n_rows // _BLK,),
        in_specs=[
            pl.BlockSpec((_BLK, _EMBED), lambda j: (j, 0)),
            pl.BlockSpec((s_len, _EMBED), lambda j: (0, 0)),
            pl.BlockSpec((_BLK, 1), lambda j: (j, 0)),
            pl.BlockSpec((8, _EMBED), lambda j: (0, 0)),
            pl.BlockSpec((_EMBED, _DMODEL), lambda j: (0, 0)),
            pl.BlockSpec((3, _DMODEL), lambda j: (0, 0)),
        ],
        out_specs=pl.BlockSpec((_BLK, _DMODEL), lambda j: (j, 0)),
        out_shape=jax.ShapeDtypeStruct((n_rows, _DMODEL), jnp.float32),
    )(g, pe, seg, segtab, W, bgb)


def kernel(sequence, sequence_segment, token_table, seg_table, W, b, gamma, beta):
    bsz, s_len = sequence.shape
    n_rows = bsz * s_len

    g = _sc_gather(token_table, sequence.astype(jnp.int32), n_rows, s_len)

    pe = jnp.asarray(_sinusoidal_pe_np(_MAXLEN, _EMBED)[:s_len])
    bgb = jnp.stack([b, gamma, beta])

    segtab_pad = jnp.zeros((8, _EMBED), jnp.float32).at[:3].set(seg_table)
    seg_col = jnp.reshape(sequence_segment.astype(jnp.int8), (n_rows, 1))
    out = _tc_fused(g, pe, seg_col, segtab_pad, W,
                    bgb, bsz, n_rows, s_len)
    return jnp.reshape(out, (bsz, s_len, _DMODEL))
